# baseline (device time: 23939 ns/iter reference)
import jax
import jax.numpy as jnp
from jax import lax
from jax.experimental import pallas as pl
from jax.experimental.pallas import tpu as pltpu

N_DEV = 4
B = 2
SQ = 128
HQ = 4
DH = 64
WINDOW = 128
SCALE = 0.125


def kernel(x, Wq, K_ext, V_ext, Wo):
    def body(x_ref, wq_ref, k_ref, v_ref, wo_ref, out_ref,
             k_buf, v_buf, send_sems, recv_sems):
        my = lax.axis_index("i")
        left = lax.rem(my + N_DEV - 1, N_DEV)
        right = lax.rem(my + 1, N_DEV)

        barrier_sem = pltpu.get_barrier_semaphore()
        for nbr in (left, right):
            pl.semaphore_signal(
                barrier_sem, inc=1,
                device_id=(nbr,), device_id_type=pl.DeviceIdType.MESH,
            )
        pl.semaphore_wait(barrier_sem, 2)

        rdmas = []
        for idx, (src, buf, slot, dst_dev) in enumerate((
            (k_ref, k_buf, 0, right),
            (v_ref, v_buf, 0, right),
            (k_ref, k_buf, 1, left),
            (v_ref, v_buf, 1, left),
        )):
            rdma = pltpu.make_async_remote_copy(
                src_ref=src,
                dst_ref=buf.at[slot],
                send_sem=send_sems.at[idx],
                recv_sem=recv_sems.at[idx],
                device_id=(dst_dev,),
                device_id_type=pl.DeviceIdType.MESH,
            )
            rdma.start()
            rdmas.append(rdma)

        q = [jnp.dot(x_ref[b], wq_ref[:, :], preferred_element_type=jnp.float32)
             for b in range(B)]

        for rdma in rdmas:
            rdma.wait()

        qi = my * SQ + lax.broadcasted_iota(jnp.int32, (SQ, 3 * SQ), 0)
        kcol = lax.broadcasted_iota(jnp.int32, (SQ, 3 * SQ), 1)
        klocal = lax.rem(kcol, SQ)
        kslot = kcol // SQ
        kbase = jnp.where(kslot == 0, left, jnp.where(kslot == 1, my, right))
        ki = kbase * SQ + klocal
        neg = jnp.abs(qi - ki) > WINDOW

        for b in range(B):
            ctx_h = []
            for h in range(HQ):
                q_bh = q[b][:, h * DH:(h + 1) * DH]
                k_all = jnp.concatenate(
                    [k_buf[0, b, :, h, :], k_ref[b, :, h, :],
                     k_buf[1, b, :, h, :]], axis=0)
                v_all = jnp.concatenate(
                    [v_buf[0, b, :, h, :], v_ref[b, :, h, :],
                     v_buf[1, b, :, h, :]], axis=0)
                scores = lax.dot_general(
                    q_bh, k_all, (((1,), (1,)), ((), ())),
                    preferred_element_type=jnp.float32) * SCALE
                scores = jnp.where(neg, -1e9, scores)
                m = jnp.max(scores, axis=-1, keepdims=True)
                w = jnp.exp(scores - m)
                w = w / jnp.sum(w, axis=-1, keepdims=True)
                ctx_h.append(jnp.dot(w, v_all,
                                     preferred_element_type=jnp.float32))
            ctx = jnp.concatenate(ctx_h, axis=1)
            out_ref[b, :, :] = jnp.dot(ctx, wo_ref[:, :],
                                       preferred_element_type=jnp.float32)

    out_shape = jax.ShapeDtypeStruct((B, SQ, 512), jnp.float32)
    return pl.pallas_call(
        body,
        out_shape=out_shape,
        in_specs=[pl.BlockSpec(memory_space=pltpu.VMEM)] * 5,
        out_specs=pl.BlockSpec(memory_space=pltpu.VMEM),
        scratch_shapes=[
            pltpu.VMEM((2, B, SQ, HQ, DH), jnp.float32),
            pltpu.VMEM((2, B, SQ, HQ, DH), jnp.float32),
            pltpu.SemaphoreType.DMA((4,)),
            pltpu.SemaphoreType.DMA((4,)),
        ],
        compiler_params=pltpu.CompilerParams(collective_id=0),
    )(x, Wq, K_ext, V_ext, Wo)


# device time: 18391 ns/iter; 1.3017x vs baseline; 1.3017x over previous
import jax
import jax.numpy as jnp
from jax import lax
from jax.experimental import pallas as pl
from jax.experimental.pallas import tpu as pltpu

N_DEV = 4
B = 2
SQ = 128
HQ = 4
DH = 64
WINDOW = 128
SCALE = 0.125


def kernel(x, Wq, K_ext, V_ext, Wo):
    def body(x_ref, wq_ref, k_ref, v_ref, wo_ref, out_ref,
             k_buf, v_buf, send_sems, recv_sems):
        my = lax.axis_index("i")
        left = lax.rem(my + N_DEV - 1, N_DEV)
        right = lax.rem(my + 1, N_DEV)

        barrier_sem = pltpu.get_barrier_semaphore()
        for nbr in (left, right):
            pl.semaphore_signal(
                barrier_sem, inc=1,
                device_id=(nbr,), device_id_type=pl.DeviceIdType.MESH,
            )
        pl.semaphore_wait(barrier_sem, 2)

        rdmas = []
        for idx, (src, buf, slot, dst_dev) in enumerate((
            (k_ref, k_buf, 0, right),
            (v_ref, v_buf, 0, right),
            (k_ref, k_buf, 1, left),
            (v_ref, v_buf, 1, left),
        )):
            rdma = pltpu.make_async_remote_copy(
                src_ref=src,
                dst_ref=buf.at[slot],
                send_sem=send_sems.at[idx],
                recv_sem=recv_sems.at[idx],
                device_id=(dst_dev,),
                device_id_type=pl.DeviceIdType.MESH,
            )
            rdma.start()
            rdmas.append(rdma)

        q = [jnp.dot(x_ref[b], wq_ref[:, :], preferred_element_type=jnp.float32)
             for b in range(B)]

        for rdma in rdmas:
            rdma.wait()

        qi = my * SQ + lax.broadcasted_iota(jnp.int32, (SQ, 3 * SQ), 0)
        kcol = lax.broadcasted_iota(jnp.int32, (SQ, 3 * SQ), 1)
        klocal = lax.rem(kcol, SQ)
        kslot = kcol // SQ
        kbase = jnp.where(kslot == 0, left, jnp.where(kslot == 1, my, right))
        ki = kbase * SQ + klocal
        neg = jnp.abs(qi - ki) > WINDOW

        for b in range(B):
            ctx_h = []
            for h in range(HQ):
                q_bh = q[b][:, h * DH:(h + 1) * DH]
                k_all = jnp.concatenate(
                    [k_buf[0, b, :, h * DH:(h + 1) * DH],
                     k_ref[b, :, h * DH:(h + 1) * DH],
                     k_buf[1, b, :, h * DH:(h + 1) * DH]], axis=0)
                v_all = jnp.concatenate(
                    [v_buf[0, b, :, h * DH:(h + 1) * DH],
                     v_ref[b, :, h * DH:(h + 1) * DH],
                     v_buf[1, b, :, h * DH:(h + 1) * DH]], axis=0)
                scores = lax.dot_general(
                    q_bh, k_all, (((1,), (1,)), ((), ())),
                    preferred_element_type=jnp.float32) * SCALE
                scores = jnp.where(neg, -1e9, scores)
                m = jnp.max(scores, axis=-1, keepdims=True)
                w = jnp.exp(scores - m)
                w = w / jnp.sum(w, axis=-1, keepdims=True)
                ctx_h.append(jnp.dot(w, v_all,
                                     preferred_element_type=jnp.float32))
            ctx = jnp.concatenate(ctx_h, axis=1)
            out_ref[b, :, :] = jnp.dot(ctx, wo_ref[:, :],
                                       preferred_element_type=jnp.float32)

    out_shape = jax.ShapeDtypeStruct((B, SQ, 512), jnp.float32)
    return pl.pallas_call(
        body,
        out_shape=out_shape,
        in_specs=[pl.BlockSpec(memory_space=pltpu.VMEM)] * 5,
        out_specs=pl.BlockSpec(memory_space=pltpu.VMEM),
        scratch_shapes=[
            pltpu.VMEM((2, B, SQ, HQ * DH), jnp.float32),
            pltpu.VMEM((2, B, SQ, HQ * DH), jnp.float32),
            pltpu.SemaphoreType.DMA((4,)),
            pltpu.SemaphoreType.DMA((4,)),
        ],
        compiler_params=pltpu.CompilerParams(collective_id=0),
    )(x, Wq, K_ext.reshape(B, SQ, HQ * DH), V_ext.reshape(B, SQ, HQ * DH), Wo)


# device time: 12932 ns/iter; 1.8511x vs baseline; 1.4221x over previous
import jax
import jax.numpy as jnp
from jax import lax
from jax.experimental import pallas as pl
from jax.experimental.pallas import tpu as pltpu

N_DEV = 4
B = 2
SQ = 128
HQ = 4
DH = 64
D = HQ * DH
WINDOW = 128
SCALE = 0.125


def kernel(x, Wq, K_ext, V_ext, Wo):
    def body(x_ref, wq_ref, k_ref, v_ref, wo_ref, out_ref,
             k_buf, v_buf, send_sems, recv_sems):
        my = lax.axis_index("i")
        left = lax.rem(my + N_DEV - 1, N_DEV)
        right = lax.rem(my + 1, N_DEV)

        barrier_sem = pltpu.get_barrier_semaphore()
        for nbr in (left, right):
            pl.semaphore_signal(
                barrier_sem, inc=1,
                device_id=(nbr,), device_id_type=pl.DeviceIdType.MESH,
            )
        pl.semaphore_wait(barrier_sem, 2)

        rdmas = []
        for idx, (src, buf, slot, dst_dev) in enumerate((
            (k_ref, k_buf, 0, right),
            (k_ref, k_buf, 1, left),
            (v_ref, v_buf, 0, right),
            (v_ref, v_buf, 1, left),
        )):
            rdma = pltpu.make_async_remote_copy(
                src_ref=src,
                dst_ref=buf.at[slot],
                send_sem=send_sems.at[idx],
                recv_sem=recv_sems.at[idx],
                device_id=(dst_dev,),
                device_id_type=pl.DeviceIdType.MESH,
            )
            rdma.start()
            rdmas.append(rdma)

        q = [jnp.dot(x_ref[b], wq_ref[:, :],
                     preferred_element_type=jnp.float32).astype(jnp.bfloat16)
             for b in range(B)]

        rdmas[0].wait()
        rdmas[1].wait()

        qi = my * SQ + lax.broadcasted_iota(jnp.int32, (SQ, 3 * SQ), 0)
        kcol = lax.broadcasted_iota(jnp.int32, (SQ, 3 * SQ), 1)
        klocal = lax.rem(kcol, SQ)
        kslot = kcol // SQ
        kbase = jnp.where(kslot == 0, left, jnp.where(kslot == 1, my, right))
        ki = kbase * SQ + klocal
        neg = jnp.abs(qi - ki) > WINDOW

        weights = []
        for b in range(B):
            for h in range(HQ):
                q_bh = q[b][:, h * DH:(h + 1) * DH]
                k_all = jnp.concatenate(
                    [k_buf[0, b, :, h * DH:(h + 1) * DH],
                     k_ref[b, :, h * DH:(h + 1) * DH],
                     k_buf[1, b, :, h * DH:(h + 1) * DH]], axis=0)
                scores = lax.dot_general(
                    q_bh, k_all, (((1,), (1,)), ((), ())),
                    preferred_element_type=jnp.float32) * SCALE
                scores = jnp.where(neg, -1e9, scores)
                m = jnp.max(scores, axis=-1, keepdims=True)
                w = jnp.exp(scores - m)
                w = w / jnp.sum(w, axis=-1, keepdims=True)
                weights.append(w.astype(jnp.bfloat16))

        rdmas[2].wait()
        rdmas[3].wait()

        for b in range(B):
            ctx_h = []
            for h in range(HQ):
                v_all = jnp.concatenate(
                    [v_buf[0, b, :, h * DH:(h + 1) * DH],
                     v_ref[b, :, h * DH:(h + 1) * DH],
                     v_buf[1, b, :, h * DH:(h + 1) * DH]], axis=0)
                ctx_h.append(jnp.dot(weights[b * HQ + h], v_all,
                                     preferred_element_type=jnp.float32))
            ctx = jnp.concatenate(ctx_h, axis=1).astype(jnp.bfloat16)
            out_ref[b, :, :] = jnp.dot(ctx, wo_ref[:, :],
                                       preferred_element_type=jnp.float32)

    out_shape = jax.ShapeDtypeStruct((B, SQ, 512), jnp.float32)
    bf16 = jnp.bfloat16
    return pl.pallas_call(
        body,
        out_shape=out_shape,
        in_specs=[pl.BlockSpec(memory_space=pltpu.VMEM)] * 5,
        out_specs=pl.BlockSpec(memory_space=pltpu.VMEM),
        scratch_shapes=[
            pltpu.VMEM((2, B, SQ, D), bf16),
            pltpu.VMEM((2, B, SQ, D), bf16),
            pltpu.SemaphoreType.DMA((4,)),
            pltpu.SemaphoreType.DMA((4,)),
        ],
        compiler_params=pltpu.CompilerParams(collective_id=0),
    )(
        x.astype(bf16),
        Wq.astype(bf16),
        K_ext.reshape(B, SQ, D).astype(bf16),
        V_ext.reshape(B, SQ, D).astype(bf16),
        Wo.astype(bf16),
    )


# device time: 12540 ns/iter; 1.9090x vs baseline; 1.0313x over previous
import jax
import jax.numpy as jnp
from jax import lax
from jax.experimental import pallas as pl
from jax.experimental.pallas import tpu as pltpu

N_DEV = 4
B = 2
SQ = 128
HQ = 4
DH = 64
D = HQ * DH
WINDOW = 128
SCALE = 0.125
BF16 = jnp.bfloat16


def kernel(x, Wq, K_ext, V_ext, Wo):
    def body(x_ref, wq_ref, k_ref, v_ref, wo_ref, out_ref,
             k_buf, v_buf, send_sems, recv_sems):
        my = lax.axis_index("i")
        left = lax.rem(my + N_DEV - 1, N_DEV)
        right = lax.rem(my + 1, N_DEV)

        barrier_sem = pltpu.get_barrier_semaphore()
        for nbr in (left, right):
            pl.semaphore_signal(
                barrier_sem, inc=1,
                device_id=(nbr,), device_id_type=pl.DeviceIdType.MESH,
            )
        pl.semaphore_wait(barrier_sem, 2)

        rdmas = []
        for idx, (src, buf, slot, dst_dev) in enumerate((
            (k_ref, k_buf, 0, right),
            (k_ref, k_buf, 1, left),
            (v_ref, v_buf, 0, right),
            (v_ref, v_buf, 1, left),
        )):
            rdma = pltpu.make_async_remote_copy(
                src_ref=src,
                dst_ref=buf.at[slot],
                send_sem=send_sems.at[idx],
                recv_sem=recv_sems.at[idx],
                device_id=(dst_dev,),
                device_id_type=pl.DeviceIdType.MESH,
            )
            rdma.start()
            rdmas.append(rdma)

        q_all = jnp.dot(x_ref[...].reshape(B * SQ, 512), wq_ref[...],
                        preferred_element_type=jnp.float32)
        q_all = q_all.astype(BF16)

        m0s, l0s, acc0s, qs = [], [], [], []
        for b in range(B):
            for h in range(HQ):
                q_bh = q_all[b * SQ:(b + 1) * SQ, h * DH:(h + 1) * DH]
                qs.append(q_bh)
                s_loc = lax.dot_general(
                    q_bh, k_ref[b, :, h * DH:(h + 1) * DH],
                    (((1,), (1,)), ((), ())),
                    preferred_element_type=jnp.float32) * SCALE
                m0 = jnp.max(s_loc, axis=-1, keepdims=True)
                w0 = jnp.exp(s_loc - m0)
                m0s.append(m0)
                l0s.append(jnp.sum(w0, axis=-1, keepdims=True))
                acc0s.append(jnp.dot(w0.astype(BF16),
                                     v_ref[b, :, h * DH:(h + 1) * DH],
                                     preferred_element_type=jnp.float32))

        qi = my * SQ + lax.broadcasted_iota(jnp.int32, (SQ, 2 * SQ), 0)
        kcol = lax.broadcasted_iota(jnp.int32, (SQ, 2 * SQ), 1)
        kbase = jnp.where(kcol < SQ, left, right)
        ki = kbase * SQ + lax.rem(kcol, SQ)
        masked = jnp.abs(qi - ki) > WINDOW

        rdmas[0].wait()
        rdmas[1].wait()

        ws, ms, alphas = [], [], []
        for b in range(B):
            k_hal = jnp.concatenate(
                [k_buf[0, b], k_buf[1, b]], axis=0)
            for h in range(HQ):
                i = b * HQ + h
                s_hal = lax.dot_general(
                    qs[i], k_hal[:, h * DH:(h + 1) * DH],
                    (((1,), (1,)), ((), ())),
                    preferred_element_type=jnp.float32) * SCALE
                s_hal = jnp.where(masked, -1e9, s_hal)
                m = jnp.maximum(m0s[i], jnp.max(s_hal, axis=-1, keepdims=True))
                w_hal = jnp.exp(s_hal - m)
                ws.append(w_hal.astype(BF16))
                ms.append(jnp.sum(w_hal, axis=-1, keepdims=True))
                alphas.append(jnp.exp(m0s[i] - m))

        rdmas[2].wait()
        rdmas[3].wait()

        ctxs = []
        for b in range(B):
            v_hal = jnp.concatenate(
                [v_buf[0, b], v_buf[1, b]], axis=0)
            for h in range(HQ):
                i = b * HQ + h
                acc = alphas[i] * acc0s[i] + jnp.dot(
                    ws[i], v_hal[:, h * DH:(h + 1) * DH],
                    preferred_element_type=jnp.float32)
                l = alphas[i] * l0s[i] + ms[i]
                ctxs.append((acc / l).astype(BF16))

        ctx_all = jnp.concatenate(
            [jnp.concatenate(ctxs[b * HQ:(b + 1) * HQ], axis=1)
             for b in range(B)], axis=0)
        out = jnp.dot(ctx_all, wo_ref[...],
                      preferred_element_type=jnp.float32)
        out_ref[...] = out.reshape(B, SQ, 512)

    out_shape = jax.ShapeDtypeStruct((B, SQ, 512), jnp.float32)
    return pl.pallas_call(
        body,
        out_shape=out_shape,
        in_specs=[pl.BlockSpec(memory_space=pltpu.VMEM)] * 5,
        out_specs=pl.BlockSpec(memory_space=pltpu.VMEM),
        scratch_shapes=[
            pltpu.VMEM((2, B, SQ, D), BF16),
            pltpu.VMEM((2, B, SQ, D), BF16),
            pltpu.SemaphoreType.DMA((4,)),
            pltpu.SemaphoreType.DMA((4,)),
        ],
        compiler_params=pltpu.CompilerParams(collective_id=0),
    )(
        x.astype(BF16),
        Wq.astype(BF16),
        K_ext.reshape(B, SQ, D).astype(BF16),
        V_ext.reshape(B, SQ, D).astype(BF16),
        Wo.astype(BF16),
    )


# device time: 12396 ns/iter; 1.9312x vs baseline; 1.0116x over previous
import jax
import jax.numpy as jnp
from jax import lax
from jax.experimental import pallas as pl
from jax.experimental.pallas import tpu as pltpu

N_DEV = 4
B = 2
SQ = 128
HQ = 4
DH = 64
D = HQ * DH
WINDOW = 128
SCALE = 0.125
BF16 = jnp.bfloat16


def kernel(x, Wq, K_ext, V_ext, Wo):
    def body(x_ref, wq_ref, k_ref, v_ref, wo_ref, out_ref,
             k_buf, v_buf, send_sems, recv_sems):
        my = lax.axis_index("i")
        left = lax.rem(my + N_DEV - 1, N_DEV)
        right = lax.rem(my + 1, N_DEV)

        barrier_sem = pltpu.get_barrier_semaphore()
        for nbr in (left, right):
            pl.semaphore_signal(
                barrier_sem, inc=1,
                device_id=(nbr,), device_id_type=pl.DeviceIdType.MESH,
            )
        pl.semaphore_wait(barrier_sem, 2)

        rdmas = []
        for idx, (src, buf, slot, dst_dev) in enumerate((
            (k_ref, k_buf, 0, right),
            (k_ref, k_buf, 1, left),
            (v_ref, v_buf, 0, right),
            (v_ref, v_buf, 1, left),
        )):
            rdma = pltpu.make_async_remote_copy(
                src_ref=src,
                dst_ref=buf.at[slot],
                send_sem=send_sems.at[idx],
                recv_sem=recv_sems.at[idx],
                device_id=(dst_dev,),
                device_id_type=pl.DeviceIdType.MESH,
            )
            rdma.start()
            rdmas.append(rdma)

        q_all = jnp.dot(x_ref[...].reshape(B * SQ, 512), wq_ref[...],
                        preferred_element_type=jnp.float32)
        q_all = q_all.astype(BF16)

        m0s, l0s, acc0s, qs = [], [], [], []
        for b in range(B):
            for h in range(HQ):
                q_bh = q_all[b * SQ:(b + 1) * SQ, h * DH:(h + 1) * DH]
                qs.append(q_bh)
                s_loc = lax.dot_general(
                    q_bh, k_ref[b, :, h * DH:(h + 1) * DH],
                    (((1,), (1,)), ((), ())),
                    preferred_element_type=jnp.float32) * SCALE
                m0 = jnp.max(s_loc, axis=-1, keepdims=True)
                w0 = jnp.exp(s_loc - m0)
                m0s.append(m0)
                l0s.append(jnp.sum(w0, axis=-1, keepdims=True))
                acc0s.append(jnp.dot(w0.astype(BF16),
                                     v_ref[b, :, h * DH:(h + 1) * DH],
                                     preferred_element_type=jnp.float32))

        qi = my * SQ + lax.broadcasted_iota(jnp.int32, (SQ, 2 * SQ), 0)
        kcol = lax.broadcasted_iota(jnp.int32, (SQ, 2 * SQ), 1)
        kbase = jnp.where(kcol < SQ, left, right)
        ki = kbase * SQ + lax.rem(kcol, SQ)
        masked = jnp.abs(qi - ki) > WINDOW

        rdmas[0].wait()
        rdmas[1].wait()

        ws, ms, alphas = [], [], []
        for b in range(B):
            k_hal = jnp.concatenate(
                [k_buf[0, b], k_buf[1, b]], axis=0)
            for h in range(HQ):
                i = b * HQ + h
                s_hal = lax.dot_general(
                    qs[i], k_hal[:, h * DH:(h + 1) * DH],
                    (((1,), (1,)), ((), ())),
                    preferred_element_type=jnp.float32) * SCALE
                s_hal = jnp.where(masked, -1e9, s_hal)
                m = jnp.maximum(m0s[i], jnp.max(s_hal, axis=-1, keepdims=True))
                w_hal = jnp.exp(s_hal - m)
                ws.append(w_hal.astype(BF16))
                ms.append(jnp.sum(w_hal, axis=-1, keepdims=True))
                alphas.append(jnp.exp(m0s[i] - m))

        rdmas[2].wait()
        rdmas[3].wait()

        ctxs = []
        for b in range(B):
            v_hal = jnp.concatenate(
                [v_buf[0, b], v_buf[1, b]], axis=0)
            for h in range(HQ):
                i = b * HQ + h
                acc = alphas[i] * acc0s[i] + jnp.dot(
                    ws[i], v_hal[:, h * DH:(h + 1) * DH],
                    preferred_element_type=jnp.float32)
                l = alphas[i] * l0s[i] + ms[i]
                ctxs.append((acc / l).astype(BF16))

        ctx_all = jnp.concatenate(
            [jnp.concatenate(ctxs[b * HQ:(b + 1) * HQ], axis=1)
             for b in range(B)], axis=0)
        out = jnp.dot(ctx_all, wo_ref[...],
                      preferred_element_type=jnp.float32)
        out_ref[...] = out.astype(BF16).reshape(B, SQ, 512)

    out_shape = jax.ShapeDtypeStruct((B, SQ, 512), BF16)
    return pl.pallas_call(
        body,
        out_shape=out_shape,
        in_specs=[pl.BlockSpec(memory_space=pltpu.VMEM)] * 5,
        out_specs=pl.BlockSpec(memory_space=pltpu.VMEM),
        scratch_shapes=[
            pltpu.VMEM((2, B, SQ, D), BF16),
            pltpu.VMEM((2, B, SQ, D), BF16),
            pltpu.SemaphoreType.DMA((4,)),
            pltpu.SemaphoreType.DMA((4,)),
        ],
        compiler_params=pltpu.CompilerParams(collective_id=0),
    )(
        x.astype(BF16),
        Wq.astype(BF16),
        K_ext.reshape(B, SQ, D).astype(BF16),
        V_ext.reshape(B, SQ, D).astype(BF16),
        Wo.astype(BF16),
    )
